# Initial kernel scaffold; baseline (speedup 1.0000x reference)
import jax
import jax.numpy as jnp
from jax.experimental import pallas as pl


def kernel(projections, localPrototypes, glu_W, glu_b):
    raise NotImplementedError("write your pallas kernel here")


# speedup vs baseline: 1.5178x; 1.5178x over previous
import functools

import jax
import jax.numpy as jnp
from jax import lax
from jax.experimental import pallas as pl
from jax.experimental.pallas import tpu as pltpu
from jax.experimental.pallas import tpu_sc as plsc

N_TOK = 9216
D = 768
P = 1024
BLK = 512
NBLK = N_TOK // BLK
INV_TEMP = 20.0
EPS = 1e-12


def _norm_protos_body(w_ref, out_ref):
    w = w_ref[...]
    sq = jnp.sum(w * w, axis=1, keepdims=True)
    out_ref[...] = w * lax.rsqrt(jnp.maximum(sq, EPS))


def _k2_body(x_ref, mn_ref, e_ref, r_ref, t_ref):
    i = pl.program_id(0)
    x = x_ref[...]
    sq = jnp.sum(x * x, axis=1, keepdims=True)
    xn = x * lax.rsqrt(jnp.maximum(sq, EPS))
    mm = lax.dot_general(xn, mn_ref[...], (((1,), (1,)), ((), ())),
                         preferred_element_type=jnp.float32)
    e = jnp.exp(mm * INV_TEMP)
    e_ref[...] = e
    r = 1.0 / (jnp.float32(N_TOK) * jnp.sum(e, axis=1, keepdims=True))
    r_ref[...] = r

    @pl.when(i == 0)
    def _():
        t_ref[...] = jnp.zeros_like(t_ref)

    t_ref[...] += jnp.sum(e * r, axis=0, keepdims=True)


def _k3_body(e_ref, tp_ref, t_ref):
    i = pl.program_id(0)
    e = e_ref[...]
    c = 1.0 / (jnp.float32(P) * tp_ref[...])
    ec = jnp.sum(e * c, axis=1, keepdims=True)
    r = 1.0 / (jnp.float32(N_TOK) * ec)

    @pl.when(i == 0)
    def _():
        t_ref[...] = jnp.zeros_like(t_ref)

    t_ref[...] += jnp.sum(e * r, axis=0, keepdims=True)


def _k4_body(e_ref, tp_ref, idx_ref):
    c = 1.0 / (jnp.float32(P) * tp_ref[...])
    scores = e_ref[...] * c
    idx_ref[...] = jnp.argmax(scores, axis=1).astype(jnp.int32)[:, None]


def _k5_body(x_ref, g_ref, wa_ref, wb_ref, ba_ref, bb_ref, out_ref):
    comb = 0.5 * x_ref[...] + 0.5 * g_ref[...]
    lin_a = lax.dot_general(comb, wa_ref[...], (((1,), (0,)), ((), ())),
                            preferred_element_type=jnp.float32) + ba_ref[...]
    lin_b = lax.dot_general(comb, wb_ref[...], (((1,), (0,)), ((), ())),
                            preferred_element_type=jnp.float32) + bb_ref[...]
    out_ref[...] = lin_a * (1.0 / (1.0 + jnp.exp(-lin_b)))


_GB = 96


def _sc_gather(table, idx):
    info = plsc.get_sparse_core_info()
    nw = info.num_cores * info.num_subcores
    b_per_w = N_TOK // nw
    nchunk = b_per_w // _GB
    mesh = plsc.VectorSubcoreMesh(core_axis_name="c", subcore_axis_name="s")

    @functools.partial(
        pl.kernel, mesh=mesh,
        out_type=jax.ShapeDtypeStruct((N_TOK, D), jnp.float32),
        scratch_types=[
            pltpu.VMEM((_GB,), jnp.int32),
            pltpu.VMEM((_GB, D), jnp.float32),
            pltpu.SemaphoreType.DMA,
        ],
    )
    def gather_k(table_hbm, idx_hbm, out_hbm, idx_v, rows_v, sem):
        wid = lax.axis_index("s") * info.num_cores + lax.axis_index("c")
        base = wid * b_per_w
        for chunk in range(nchunk):
            off = base + chunk * _GB
            pltpu.sync_copy(idx_hbm.at[pl.ds(off, _GB)], idx_v)
            pltpu.async_copy(table_hbm.at[idx_v], rows_v, sem).wait()
            pltpu.sync_copy(rows_v, out_hbm.at[pl.ds(off, _GB)])

    return gather_k(table, idx)


def kernel(projections, localPrototypes, glu_W, glu_b):
    shp = projections.shape
    flat = projections.reshape(N_TOK, D)

    mem_n = pl.pallas_call(
        _norm_protos_body,
        out_shape=jax.ShapeDtypeStruct((P, D), jnp.float32),
    )(localPrototypes)

    e, _r1, t1 = pl.pallas_call(
        _k2_body,
        grid=(NBLK,),
        in_specs=[
            pl.BlockSpec((BLK, D), lambda i: (i, 0)),
            pl.BlockSpec((P, D), lambda i: (0, 0)),
        ],
        out_specs=[
            pl.BlockSpec((BLK, P), lambda i: (i, 0)),
            pl.BlockSpec((BLK, 1), lambda i: (i, 0)),
            pl.BlockSpec((1, P), lambda i: (0, 0)),
        ],
        out_shape=[
            jax.ShapeDtypeStruct((N_TOK, P), jnp.float32),
            jax.ShapeDtypeStruct((N_TOK, 1), jnp.float32),
            jax.ShapeDtypeStruct((1, P), jnp.float32),
        ],
    )(flat, mem_n)

    sink_iter = pl.pallas_call(
        _k3_body,
        grid=(NBLK,),
        in_specs=[
            pl.BlockSpec((BLK, P), lambda i: (i, 0)),
            pl.BlockSpec((1, P), lambda i: (0, 0)),
        ],
        out_specs=pl.BlockSpec((1, P), lambda i: (0, 0)),
        out_shape=jax.ShapeDtypeStruct((1, P), jnp.float32),
    )
    t2 = sink_iter(e, t1)
    t3 = sink_iter(e, t2)

    idx = pl.pallas_call(
        _k4_body,
        grid=(NBLK,),
        in_specs=[
            pl.BlockSpec((BLK, P), lambda i: (i, 0)),
            pl.BlockSpec((1, P), lambda i: (0, 0)),
        ],
        out_specs=pl.BlockSpec((BLK, 1), lambda i: (i, 0)),
        out_shape=jax.ShapeDtypeStruct((N_TOK, 1), jnp.int32),
    )(e, t3)

    gathered = _sc_gather(localPrototypes, idx.reshape(N_TOK))

    wa = glu_W[:, :D]
    wb = glu_W[:, D:]
    ba = glu_b[:D].reshape(1, D)
    bb = glu_b[D:].reshape(1, D)

    out = pl.pallas_call(
        _k5_body,
        grid=(NBLK,),
        in_specs=[
            pl.BlockSpec((BLK, D), lambda i: (i, 0)),
            pl.BlockSpec((BLK, D), lambda i: (i, 0)),
            pl.BlockSpec((D, D), lambda i: (0, 0)),
            pl.BlockSpec((D, D), lambda i: (0, 0)),
            pl.BlockSpec((1, D), lambda i: (0, 0)),
            pl.BlockSpec((1, D), lambda i: (0, 0)),
        ],
        out_specs=pl.BlockSpec((BLK, D), lambda i: (i, 0)),
        out_shape=jax.ShapeDtypeStruct((N_TOK, D), jnp.float32),
    )(flat, gathered, wa, wb, ba, bb)

    return out.reshape(shp)


# speedup vs baseline: 2.0584x; 1.3562x over previous
import functools

import jax
import jax.numpy as jnp
from jax import lax
from jax.experimental import pallas as pl
from jax.experimental.pallas import tpu as pltpu
from jax.experimental.pallas import tpu_sc as plsc

N_TOK = 9216
D = 768
P = 1024
BLK = 512
NBLK = N_TOK // BLK
INV_TEMP = 20.0
EPS = 1e-12


def _mega_body(x_ref, protos_ref, idx_ref, mn_s, e_s, ta_s, tb_s):
    j = pl.program_id(0)
    i = pl.program_id(1)

    @pl.when(j == 0)
    def _():
        @pl.when(i == 0)
        def _():
            w = protos_ref[...]
            sq = jnp.sum(w * w, axis=1, keepdims=True)
            mn_s[...] = w * lax.rsqrt(jnp.maximum(sq, EPS))
            ta_s[...] = jnp.zeros_like(ta_s)

        x = x_ref[...]
        sq = jnp.sum(x * x, axis=1, keepdims=True)
        xn = x * lax.rsqrt(jnp.maximum(sq, EPS))
        mm = lax.dot_general(xn, mn_s[...], (((1,), (1,)), ((), ())),
                             preferred_element_type=jnp.float32)
        e = jnp.exp(mm * INV_TEMP)
        e_s[pl.ds(i * BLK, BLK), :] = e
        r = 1.0 / (jnp.float32(N_TOK) * jnp.sum(e, axis=1, keepdims=True))
        ta_s[...] += jnp.sum(e * r, axis=0, keepdims=True)

    @pl.when(j == 1)
    def _():
        @pl.when(i == 0)
        def _():
            tb_s[...] = jnp.zeros_like(tb_s)

        e = e_s[pl.ds(i * BLK, BLK), :]
        c = 1.0 / (jnp.float32(P) * ta_s[...])
        r = 1.0 / (jnp.float32(N_TOK) * jnp.sum(e * c, axis=1, keepdims=True))
        tb_s[...] += jnp.sum(e * r, axis=0, keepdims=True)

    @pl.when(j == 2)
    def _():
        @pl.when(i == 0)
        def _():
            ta_s[...] = jnp.zeros_like(ta_s)

        e = e_s[pl.ds(i * BLK, BLK), :]
        c = 1.0 / (jnp.float32(P) * tb_s[...])
        r = 1.0 / (jnp.float32(N_TOK) * jnp.sum(e * c, axis=1, keepdims=True))
        ta_s[...] += jnp.sum(e * r, axis=0, keepdims=True)

    @pl.when(j == 3)
    def _():
        e = e_s[pl.ds(i * BLK, BLK), :]
        c3 = 1.0 / (jnp.float32(P) * ta_s[...])
        am = jnp.argmax(e * c3, axis=1).astype(jnp.int32)[:, None]
        idx_ref[pl.ds(i * BLK, BLK), :] = am


def _k5_body(x_ref, g_ref, wa_ref, wb_ref, ba_ref, bb_ref, out_ref):
    comb = (0.5 * x_ref[...] + 0.5 * g_ref[...]).astype(jnp.bfloat16)
    lin_a = lax.dot_general(comb, wa_ref[...], (((1,), (0,)), ((), ())),
                            preferred_element_type=jnp.float32) + ba_ref[...]
    lin_b = lax.dot_general(comb, wb_ref[...], (((1,), (0,)), ((), ())),
                            preferred_element_type=jnp.float32) + bb_ref[...]
    out_ref[...] = lin_a * (1.0 / (1.0 + jnp.exp(-lin_b)))


_GB = 96


def _sc_gather(table, idx):
    info = plsc.get_sparse_core_info()
    nw = info.num_cores * info.num_subcores
    b_per_w = N_TOK // nw
    nchunk = b_per_w // _GB
    mesh = plsc.VectorSubcoreMesh(core_axis_name="c", subcore_axis_name="s")

    @functools.partial(
        pl.kernel, mesh=mesh,
        out_type=jax.ShapeDtypeStruct((N_TOK, D), jnp.float32),
        scratch_types=[
            pltpu.VMEM((_GB,), jnp.int32),
            pltpu.VMEM((_GB, D), jnp.float32),
            pltpu.SemaphoreType.DMA,
        ],
    )
    def gather_k(table_hbm, idx_hbm, out_hbm, idx_v, rows_v, sem):
        wid = lax.axis_index("s") * info.num_cores + lax.axis_index("c")
        base = wid * b_per_w
        for chunk in range(nchunk):
            off = base + chunk * _GB
            pltpu.sync_copy(idx_hbm.at[pl.ds(off, _GB)], idx_v)
            pltpu.async_copy(table_hbm.at[idx_v], rows_v, sem).wait()
            pltpu.sync_copy(rows_v, out_hbm.at[pl.ds(off, _GB)])

    return gather_k(table, idx)


def kernel(projections, localPrototypes, glu_W, glu_b):
    shp = projections.shape
    flat = projections.reshape(N_TOK, D)

    idx = pl.pallas_call(
        _mega_body,
        grid=(4, NBLK),
        in_specs=[
            pl.BlockSpec((BLK, D), lambda j, i: (jax.lax.select(j == 0, i, 0), 0)),
            pl.BlockSpec((P, D), lambda j, i: (0, 0)),
        ],
        out_specs=pl.BlockSpec((N_TOK, 1), lambda j, i: (0, 0)),
        out_shape=jax.ShapeDtypeStruct((N_TOK, 1), jnp.int32),
        scratch_shapes=[
            pltpu.VMEM((P, D), jnp.float32),
            pltpu.VMEM((N_TOK, P), jnp.float32),
            pltpu.VMEM((1, P), jnp.float32),
            pltpu.VMEM((1, P), jnp.float32),
        ],
        compiler_params=pltpu.CompilerParams(
            vmem_limit_bytes=100 * 1024 * 1024,
        ),
    )(flat, localPrototypes)

    gathered = _sc_gather(localPrototypes, idx.reshape(N_TOK))

    wa = glu_W[:, :D].astype(jnp.bfloat16)
    wb = glu_W[:, D:].astype(jnp.bfloat16)
    ba = glu_b[:D].reshape(1, D)
    bb = glu_b[D:].reshape(1, D)

    out = pl.pallas_call(
        _k5_body,
        grid=(NBLK,),
        in_specs=[
            pl.BlockSpec((BLK, D), lambda i: (i, 0)),
            pl.BlockSpec((BLK, D), lambda i: (i, 0)),
            pl.BlockSpec((D, D), lambda i: (0, 0)),
            pl.BlockSpec((D, D), lambda i: (0, 0)),
            pl.BlockSpec((1, D), lambda i: (0, 0)),
            pl.BlockSpec((1, D), lambda i: (0, 0)),
        ],
        out_specs=pl.BlockSpec((BLK, D), lambda i: (i, 0)),
        out_shape=jax.ShapeDtypeStruct((N_TOK, D), jnp.float32),
    )(flat, gathered, wa, wb, ba, bb)

    return out.reshape(shp)
